# baseline (speedup 1.0000x reference)
import functools

import jax
import jax.numpy as jnp
from jax import lax
from jax.experimental import pallas as pl
from jax.experimental.pallas import tpu as pltpu
from jax.experimental.pallas import tpu_sc as plsc

NC = 2
NS = 16
NW = NC * NS
LANES = 16
K = 56


@functools.lru_cache(maxsize=None)
def _build(n_rows: int, w: int):
    WO = 2 * w
    ROW_OUT = 2 * WO
    rows_per_w = n_rows // NW
    chunks = rows_per_w // K
    GRPS = w // LANES
    assert n_rows % NW == 0 and rows_per_w % K == 0 and w % LANES == 0
    assert chunks % 2 == 0 and chunks >= 4

    mesh = plsc.VectorSubcoreMesh(
        core_axis_name="c", subcore_axis_name="s",
        num_cores=NC, num_subcores=NS)

    @functools.partial(
        pl.kernel,
        out_type=jax.ShapeDtypeStruct((2 * n_rows, WO), jnp.float32),
        mesh=mesh,
        scratch_types=[
            pltpu.VMEM((2 * K, w), jnp.float32),
            pltpu.VMEM((2 * K, w), jnp.int32),
            pltpu.VMEM((2 * K, WO), jnp.float32),
            pltpu.VMEM((2 * K, WO), jnp.float32),
            pltpu.SemaphoreType.DMA,
            pltpu.SemaphoreType.DMA,
            pltpu.SemaphoreType.DMA,
            pltpu.SemaphoreType.DMA,
        ],
        compiler_params=pltpu.CompilerParams(needs_layout_passes=False),
    )
    def unpool(x_hbm, i_hbm, o_hbm, xbuf, ibuf, obuf0, obuf1,
               si0, si1, so0, so1):
        wid = lax.axis_index("s") * NC + lax.axis_index("c")
        row0 = wid * rows_per_w
        ti = lax.iota(jnp.int32, LANES) * 2
        cols = [g * (2 * LANES) + ti for g in range(GRPS)]
        cols1 = [c + 1 for c in cols]
        s_in = (si0, si1)
        s_out = (so0, so1)
        obufs = (obuf0, obuf1)

        def in_copy(t, h):
            r0 = row0 + t * K
            pltpu.async_copy(x_hbm.at[pl.ds(r0, K)],
                             xbuf.at[pl.ds(h * K, K)], s_in[h])
            pltpu.async_copy(i_hbm.at[pl.ds(r0, K)],
                             ibuf.at[pl.ds(h * K, K)], s_in[h])

        def in_wait(h):
            pltpu.make_async_copy(x_hbm.at[pl.ds(0, K)],
                                  xbuf.at[pl.ds(h * K, K)], s_in[h]).wait()
            pltpu.make_async_copy(i_hbm.at[pl.ds(0, K)],
                                  ibuf.at[pl.ds(h * K, K)], s_in[h]).wait()

        def out_copy(t, h):
            or0 = 2 * (row0 + t * K)
            pltpu.async_copy(obufs[h].at[:, pl.ds(0, 128)],
                             o_hbm.at[pl.ds(or0, 2 * K), pl.ds(0, 128)],
                             s_out[h])
            pltpu.async_copy(obufs[h].at[:, pl.ds(128, WO - 128)],
                             o_hbm.at[pl.ds(or0, 2 * K), pl.ds(128, WO - 128)],
                             s_out[h])

        def out_wait(h):
            pltpu.make_async_copy(obufs[h].at[:, pl.ds(0, 128)],
                                  o_hbm.at[pl.ds(0, 2 * K), pl.ds(0, 128)],
                                  s_out[h]).wait()
            pltpu.make_async_copy(obufs[h].at[:, pl.ds(128, WO - 128)],
                                  o_hbm.at[pl.ds(0, 2 * K), pl.ds(128, WO - 128)],
                                  s_out[h]).wait()

        def compute(h):
            ob = obufs[h]
            thr0 = h * K * ROW_OUT + WO
            zero = jnp.zeros((LANES,), jnp.float32)

            @plsc.parallel_loop(0, K, unroll=2)
            def kbody(k):
                thr = thr0 + k * ROW_OUT
                row = h * K + k
                rtop = jnp.full((LANES,), 2 * k, jnp.int32)
                rbot = jnp.full((LANES,), 2 * k + 1, jnp.int32)
                for g in range(GRPS):
                    xv = xbuf[row, pl.ds(g * LANES, LANES)]
                    iv = ibuf[row, pl.ds(g * LANES, LANES)]
                    rge = iv >= thr
                    mc1 = (iv & 1) == 1
                    vtop = jnp.where(rge, zero, xv)
                    vbot = jnp.where(rge, xv, zero)
                    v01 = jnp.where(mc1, vtop, zero)
                    v00 = vtop - v01
                    v11 = jnp.where(mc1, vbot, zero)
                    v10 = vbot - v11
                    plsc.store_scatter(ob, [rtop, cols[g]], v00)
                    plsc.store_scatter(ob, [rtop, cols1[g]], v01)
                    plsc.store_scatter(ob, [rbot, cols[g]], v10)
                    plsc.store_scatter(ob, [rbot, cols1[g]], v11)

        in_copy(0, 0)
        in_copy(1, 1)
        for h in (0, 1):
            in_wait(h)
            compute(h)
            out_copy(h, h)
            in_copy(h + 2, h)

        def ubody(u, carry):
            for h in (0, 1):
                t = 2 * u + h
                in_wait(h)
                out_wait(h)
                compute(h)
                out_copy(t, h)
                in_copy(t + 2, h)
            return carry

        lax.fori_loop(1, chunks // 2 - 1, ubody, 0)

        for t in (chunks - 2, chunks - 1):
            h = t % 2
            in_wait(h)
            out_wait(h)
            compute(h)
            out_copy(t, h)
        out_wait(0)
        out_wait(1)

    return unpool


def kernel(x, indices, output_size):
    b, t, c, h, w = x.shape
    n_rows = b * t * c * h
    fn = _build(n_rows, w)
    out = fn(x.reshape(n_rows, w), indices.reshape(n_rows, w))
    return out.reshape(b, t, c, 2 * h, 2 * w)


# speedup vs baseline: 1.0123x; 1.0123x over previous
import functools

import jax
import jax.numpy as jnp
from jax import lax
from jax.experimental import pallas as pl
from jax.experimental.pallas import tpu as pltpu
from jax.experimental.pallas import tpu_sc as plsc

NC = 2
NS = 16
NW = NC * NS
LANES = 16
K = 56


@functools.lru_cache(maxsize=None)
def _build(n_rows: int, w: int):
    WO = 2 * w
    ROW_OUT = 2 * WO
    rows_per_w = n_rows // NW
    chunks = rows_per_w // K
    GRPS = w // LANES
    assert n_rows % NW == 0 and rows_per_w % K == 0 and w % LANES == 0
    assert chunks % 2 == 0 and chunks >= 4

    mesh = plsc.VectorSubcoreMesh(
        core_axis_name="c", subcore_axis_name="s",
        num_cores=NC, num_subcores=NS)

    @functools.partial(
        pl.kernel,
        out_type=jax.ShapeDtypeStruct((2 * n_rows, WO), jnp.float32),
        mesh=mesh,
        scratch_types=[
            pltpu.VMEM((2 * K, w), jnp.float32),
            pltpu.VMEM((2 * K, w), jnp.int32),
            pltpu.VMEM((2 * K, WO), jnp.float32),
            pltpu.VMEM((2 * K, WO), jnp.float32),
            pltpu.SemaphoreType.DMA,
            pltpu.SemaphoreType.DMA,
            pltpu.SemaphoreType.DMA,
            pltpu.SemaphoreType.DMA,
        ],
        compiler_params=pltpu.CompilerParams(
            needs_layout_passes=False, skip_device_barrier=True),
    )
    def unpool(x_hbm, i_hbm, o_hbm, xbuf, ibuf, obuf0, obuf1,
               si0, si1, so0, so1):
        wid = lax.axis_index("s") * NC + lax.axis_index("c")
        row0 = wid * rows_per_w
        ti = lax.iota(jnp.int32, LANES) * 2
        cols = [g * (2 * LANES) + ti for g in range(GRPS)]
        cols1 = [c + 1 for c in cols]
        s_in = (si0, si1)
        s_out = (so0, so1)
        obufs = (obuf0, obuf1)

        def in_copy(t, h):
            r0 = row0 + t * K
            pltpu.async_copy(x_hbm.at[pl.ds(r0, K)],
                             xbuf.at[pl.ds(h * K, K)], s_in[h])
            pltpu.async_copy(i_hbm.at[pl.ds(r0, K)],
                             ibuf.at[pl.ds(h * K, K)], s_in[h])

        def in_wait(h):
            pltpu.make_async_copy(x_hbm.at[pl.ds(0, K)],
                                  xbuf.at[pl.ds(h * K, K)], s_in[h]).wait()
            pltpu.make_async_copy(i_hbm.at[pl.ds(0, K)],
                                  ibuf.at[pl.ds(h * K, K)], s_in[h]).wait()

        def out_copy(t, h):
            or0 = 2 * (row0 + t * K)
            pltpu.async_copy(obufs[h], o_hbm.at[pl.ds(or0, 2 * K)], s_out[h])

        def out_wait(h):
            pltpu.make_async_copy(obufs[h],
                                  o_hbm.at[pl.ds(0, 2 * K)], s_out[h]).wait()

        def compute(h):
            ob = obufs[h]
            thr0 = h * K * ROW_OUT + WO
            zero = jnp.zeros((LANES,), jnp.float32)

            @plsc.parallel_loop(0, K, unroll=2)
            def kbody(k):
                thr = thr0 + k * ROW_OUT
                row = h * K + k
                rtop = jnp.full((LANES,), 2 * k, jnp.int32)
                rbot = jnp.full((LANES,), 2 * k + 1, jnp.int32)
                for g in range(GRPS):
                    xv = xbuf[row, pl.ds(g * LANES, LANES)]
                    iv = ibuf[row, pl.ds(g * LANES, LANES)]
                    rge = iv >= thr
                    mc1 = (iv & 1) == 1
                    vtop = jnp.where(rge, zero, xv)
                    vbot = jnp.where(rge, xv, zero)
                    v01 = jnp.where(mc1, vtop, zero)
                    v00 = vtop - v01
                    v11 = jnp.where(mc1, vbot, zero)
                    v10 = vbot - v11
                    plsc.store_scatter(ob, [rtop, cols[g]], v00)
                    plsc.store_scatter(ob, [rtop, cols1[g]], v01)
                    plsc.store_scatter(ob, [rbot, cols[g]], v10)
                    plsc.store_scatter(ob, [rbot, cols1[g]], v11)

        in_copy(0, 0)
        in_copy(1, 1)
        for h in (0, 1):
            in_wait(h)
            compute(h)
            out_copy(h, h)
            in_copy(h + 2, h)

        def ubody(u, carry):
            for h in (0, 1):
                t = 2 * u + h
                in_wait(h)
                out_wait(h)
                compute(h)
                out_copy(t, h)
                in_copy(t + 2, h)
            return carry

        lax.fori_loop(1, chunks // 2 - 1, ubody, 0)

        for t in (chunks - 2, chunks - 1):
            h = t % 2
            in_wait(h)
            out_wait(h)
            compute(h)
            out_copy(t, h)
        out_wait(0)
        out_wait(1)

    return unpool


def kernel(x, indices, output_size):
    b, t, c, h, w = x.shape
    n_rows = b * t * c * h
    fn = _build(n_rows, w)
    out = fn(x.reshape(n_rows, w), indices.reshape(n_rows, w))
    return out.reshape(b, t, c, 2 * h, 2 * w)


# speedup vs baseline: 1.0130x; 1.0007x over previous
import functools

import jax
import jax.numpy as jnp
from jax import lax
from jax.experimental import pallas as pl
from jax.experimental.pallas import tpu as pltpu
from jax.experimental.pallas import tpu_sc as plsc

NC = 2
NS = 16
NW = NC * NS
LANES = 16
K = 56


@functools.lru_cache(maxsize=None)
def _build(n_rows: int, w: int):
    WO = 2 * w
    ROW_OUT = 2 * WO
    rows_per_w = n_rows // NW
    chunks = rows_per_w // K
    GRPS = w // LANES
    assert n_rows % NW == 0 and rows_per_w % K == 0 and w % LANES == 0
    assert chunks % 2 == 0 and chunks >= 4

    mesh = plsc.VectorSubcoreMesh(
        core_axis_name="c", subcore_axis_name="s",
        num_cores=NC, num_subcores=NS)

    @functools.partial(
        pl.kernel,
        out_type=jax.ShapeDtypeStruct((2 * n_rows, WO), jnp.float32),
        mesh=mesh,
        scratch_types=[
            pltpu.VMEM((2 * K, w), jnp.float32),
            pltpu.VMEM((2 * K, w), jnp.int32),
            pltpu.VMEM((2 * K, WO), jnp.float32),
            pltpu.VMEM((2 * K, WO), jnp.float32),
            pltpu.SemaphoreType.DMA,
            pltpu.SemaphoreType.DMA,
            pltpu.SemaphoreType.DMA,
            pltpu.SemaphoreType.DMA,
        ],
        compiler_params=pltpu.CompilerParams(needs_layout_passes=False),
    )
    def unpool(x_hbm, i_hbm, o_hbm, xbuf, ibuf, obuf0, obuf1,
               si0, si1, so0, so1):
        wid = lax.axis_index("s") * NC + lax.axis_index("c")
        row0 = wid * rows_per_w
        ti = lax.iota(jnp.int32, LANES) * 2
        cols = [g * (2 * LANES) + ti for g in range(GRPS)]
        cols1 = [c + 1 for c in cols]
        s_in = (si0, si1)
        s_out = (so0, so1)
        obufs = (obuf0, obuf1)

        def in_copy(t, h):
            r0 = row0 + t * K
            pltpu.async_copy(x_hbm.at[pl.ds(r0, K)],
                             xbuf.at[pl.ds(h * K, K)], s_in[h])
            pltpu.async_copy(i_hbm.at[pl.ds(r0, K)],
                             ibuf.at[pl.ds(h * K, K)], s_in[h])

        def in_wait(h):
            pltpu.make_async_copy(x_hbm.at[pl.ds(0, K)],
                                  xbuf.at[pl.ds(h * K, K)], s_in[h]).wait()
            pltpu.make_async_copy(i_hbm.at[pl.ds(0, K)],
                                  ibuf.at[pl.ds(h * K, K)], s_in[h]).wait()

        def out_copy(t, h):
            or0 = 2 * (row0 + t * K)
            pltpu.async_copy(obufs[h], o_hbm.at[pl.ds(or0, 2 * K)], s_out[h])

        def out_wait(h):
            pltpu.make_async_copy(obufs[h],
                                  o_hbm.at[pl.ds(0, 2 * K)], s_out[h]).wait()

        def compute(h):
            ob = obufs[h]
            thr0 = h * K * ROW_OUT + WO
            zero = jnp.zeros((LANES,), jnp.float32)

            @plsc.parallel_loop(0, K, unroll=2)
            def kbody(k):
                thr = thr0 + k * ROW_OUT
                row = h * K + k
                rtop = jnp.full((LANES,), 2 * k, jnp.int32)
                rbot = jnp.full((LANES,), 2 * k + 1, jnp.int32)
                for g in range(GRPS):
                    xv = xbuf[row, pl.ds(g * LANES, LANES)]
                    iv = ibuf[row, pl.ds(g * LANES, LANES)]
                    rge = iv >= thr
                    mc1 = (iv & 1) == 1
                    vtop = jnp.where(rge, zero, xv)
                    vbot = jnp.where(rge, xv, zero)
                    v01 = jnp.where(mc1, vtop, zero)
                    v00 = vtop - v01
                    v11 = jnp.where(mc1, vbot, zero)
                    v10 = vbot - v11
                    plsc.store_scatter(ob, [rtop, cols[g]], v00)
                    plsc.store_scatter(ob, [rtop, cols1[g]], v01)
                    plsc.store_scatter(ob, [rbot, cols[g]], v10)
                    plsc.store_scatter(ob, [rbot, cols1[g]], v11)

        in_copy(0, 0)
        in_copy(1, 1)
        for h in (0, 1):
            in_wait(h)
            compute(h)
            out_copy(h, h)
            in_copy(h + 2, h)

        def ubody(u, carry):
            for h in (0, 1):
                t = 2 * u + h
                in_wait(h)
                out_wait(h)
                compute(h)
                out_copy(t, h)
                in_copy(t + 2, h)
            return carry

        lax.fori_loop(1, chunks // 2 - 1, ubody, 0)

        for t in (chunks - 2, chunks - 1):
            h = t % 2
            in_wait(h)
            out_wait(h)
            compute(h)
            out_copy(t, h)
        out_wait(0)
        out_wait(1)

    return unpool


def kernel(x, indices, output_size):
    b, t, c, h, w = x.shape
    n_rows = b * t * c * h
    fn = _build(n_rows, w)
    out = fn(x.reshape(n_rows, w), indices.reshape(n_rows, w))
    return out.reshape(b, t, c, 2 * h, 2 * w)
